# baseline (speedup 1.0000x reference)
import functools

import jax
import jax.numpy as jnp
from jax import lax
from jax.experimental import pallas as pl
from jax.experimental.pallas import tpu as pltpu
from jax.experimental.pallas import tpu_sc as plsc

NUM_RAW = 8100
NUM_POL = 2550
BATCH = 4096

NC = 2
NS = 16
L = 16
NW = NC * NS

ROWS_PER_W = BATCH // NW
R = 4
N_CHUNKS = ROWS_PER_W // R
K_IDX = (NUM_RAW + L - 1) // L
K_OUT = (NUM_POL + L - 1) // L
SRC_PAD = K_OUT * L


def _body(x_hbm, idx_hbm, out_hbm, idx_v, src_v, in_v, out_v):
    wid = lax.axis_index("s") * NC + lax.axis_index("c")
    lane = lax.broadcasted_iota(jnp.int32, (L,), 0)

    pltpu.sync_copy(idx_hbm, idx_v.at[pl.ds(0, NUM_RAW)])
    src_v[pl.ds(SRC_PAD - L, L)] = jnp.zeros((L,), jnp.int32)

    def build_src(k, _):
        vec = idx_v[pl.ds(k * L, L)]
        j = k * L + lane
        m = (vec >= 0) & (j < NUM_RAW)
        addr = jnp.where(m, vec, 0)
        plsc.store_scatter(src_v, [addr], j, mask=m)
        return 0

    lax.fori_loop(0, K_IDX, build_src, 0)

    def chunk(i, _):
        base = wid * ROWS_PER_W + i * R
        pltpu.sync_copy(x_hbm.at[pl.ds(base, R)], in_v)

        def gather_k(k, _):
            col = src_v[pl.ds(k * L, L)]
            dst = k * L + lane
            m = dst < NUM_POL
            for r in range(R):
                row = jnp.full((L,), r, jnp.int32)
                val = plsc.load_gather(in_v, [row, col])
                plsc.store_scatter(out_v, [row, dst], val, mask=m)
            return 0

        lax.fori_loop(0, K_OUT, gather_k, 0)
        pltpu.sync_copy(out_v, out_hbm.at[pl.ds(base, R)])
        return 0

    lax.fori_loop(0, N_CHUNKS, chunk, 0)


@jax.jit
def kernel(policy_logits_8100, policy_index_array):
    idx32 = policy_index_array.astype(jnp.int32)
    mesh = plsc.VectorSubcoreMesh(
        core_axis_name="c", subcore_axis_name="s", num_cores=NC, num_subcores=NS
    )
    run = pl.kernel(
        _body,
        out_type=jax.ShapeDtypeStruct((BATCH, NUM_POL), jnp.float32),
        mesh=mesh,
        scratch_types=[
            pltpu.VMEM((K_IDX * L,), jnp.int32),
            pltpu.VMEM((SRC_PAD,), jnp.int32),
            pltpu.VMEM((R, NUM_RAW), jnp.float32),
            pltpu.VMEM((R, NUM_POL), jnp.float32),
        ],
        compiler_params=pltpu.CompilerParams(needs_layout_passes=False),
    )
    return run(policy_logits_8100, idx32)
